# baseline (speedup 1.0000x reference)
import functools

import jax
import jax.numpy as jnp
from jax import lax
from jax.experimental import pallas as pl
from jax.experimental.pallas import tpu as pltpu
from jax.experimental.pallas import tpu_sc as plsc

EMBED = 64
ROWS = 65
N = 16384 * 50
NW = 32
N_TILE = N // NW
CHUNK = 512
NCHUNK = N_TILE // CHUNK


def _sc_body(dist_hbm, table_hbm, out_hbm, dist_v, table_v, ptab_v, out_v0,
             out_v1, sem0, sem1):
    wid = lax.axis_index("s") * 2 + lax.axis_index("c")
    base = wid * N_TILE
    pltpu.sync_copy(table_hbm, table_v)
    pltpu.sync_copy(dist_hbm.at[pl.ds(base, N_TILE)], dist_v)

    def pack_body(k, c2):
        for c in range(4):
            lo = table_v[pl.ds(k * EMBED + c * 16, 16)]
            hi = table_v[pl.ds(k * EMBED + EMBED + c * 16, 16)]
            dl = hi - lo
            lob = plsc.bitcast(lo, jnp.int32)
            dlb = plsc.bitcast(dl, jnp.int32)
            w = ((dlb + 0x8000) & jnp.int32(-65536)) | (
                ((lob + 0x8000) >> 16) & 0xFFFF)
            ptab_v[pl.ds(k * EMBED + c * 16, 16)] = w
        return c2

    lax.fori_loop(0, EMBED, pack_body, 0)

    def compute_chunk(g, out_v):
        src_l = (g // 2) * 1024 + (g % 2) * 256

        def half(src, colb):
            def grp_body(j, c2):
                d = dist_v[pl.ds(src + j * 16, 16)]
                f = d * 64.0
                l = f.astype(jnp.int32)
                frac = f - l.astype(jnp.float32)
                li = l * EMBED
                for k0 in range(0, 16, 8):
                    bs = [li[k0 + t] for t in range(8)]
                    rs = [[ptab_v[pl.ds(b + c * 16, 16)]
                           for c in range(4)] for b in bs]
                    fs = [jnp.full((16,), frac[k0 + t], jnp.float32)
                          for t in range(8)]
                    for t in range(8):
                        k = k0 + t
                        for c in range(4):
                            w = rs[t][c]
                            lo = plsc.bitcast(w << 16, jnp.float32)
                            dl = plsc.bitcast(w, jnp.float32)
                            out_v[j * 16 + k,
                                  pl.ds(colb + c * 16, 16)] = (
                                lo + fs[t] * dl)
                return c2

            lax.fori_loop(0, 16, grp_body, 0)

        half(src_l, 0)
        half(src_l + 512, EMBED)

    base2 = wid * (N_TILE // 2)

    def pair_body(gg, carry):
        for buf, sem in ((out_v0, sem0), (out_v1, sem1)):
            g = gg * 2 + (0 if buf is out_v0 else 1)
            dst = out_hbm.at[pl.ds(base2 + g * 256, 256)]

            @pl.when(gg > 0)
            def _wait():
                prev = out_hbm.at[pl.ds(base2 + (g - 2) * 256, 256)]
                pltpu.make_async_copy(buf, prev, sem).wait()

            compute_chunk(g, buf)
            pltpu.async_copy(buf, dst, sem)
        return carry

    lax.fori_loop(0, NCHUNK // 2, pair_body, 0)
    last0 = out_hbm.at[pl.ds(base2 + (NCHUNK - 2) * 256, 256)]
    last1 = out_hbm.at[pl.ds(base2 + (NCHUNK - 1) * 256, 256)]
    pltpu.make_async_copy(out_v0, last0, sem0).wait()
    pltpu.make_async_copy(out_v1, last1, sem1).wait()


_sc_kernel = functools.partial(
    pl.kernel,
    out_type=jax.ShapeDtypeStruct((N // 2, 2 * EMBED), jnp.float32),
    mesh=plsc.VectorSubcoreMesh(core_axis_name="c", subcore_axis_name="s"),
    compiler_params=pltpu.CompilerParams(needs_layout_passes=False),
    scratch_types=[
        pltpu.VMEM((N_TILE,), jnp.float32),
        pltpu.VMEM((ROWS * EMBED,), jnp.float32),
        pltpu.VMEM((EMBED * EMBED,), jnp.int32),
        pltpu.VMEM((256, 2 * EMBED), jnp.float32),
        pltpu.VMEM((256, 2 * EMBED), jnp.float32),
        pltpu.SemaphoreType.DMA,
        pltpu.SemaphoreType.DMA,
    ],
)(_sc_body)

_TC_BLK = 512


def _relayout_body(x_ref, o_ref):
    x = x_ref[...]
    o_ref[0:_TC_BLK, :] = x[:, 0:EMBED]
    o_ref[_TC_BLK:2 * _TC_BLK, :] = x[:, EMBED:2 * EMBED]


_tc_relayout = pl.pallas_call(
    _relayout_body,
    grid=(N // 2 // _TC_BLK,),
    in_specs=[pl.BlockSpec((_TC_BLK, 2 * EMBED), lambda i: (i, 0))],
    out_specs=pl.BlockSpec((2 * _TC_BLK, EMBED), lambda i: (i, 0)),
    out_shape=jax.ShapeDtypeStruct((N, EMBED), jnp.float32),
)


def kernel(dist, embed_q_weight):
    d = dist.reshape(-1).astype(jnp.float32)
    t = embed_q_weight.reshape(-1)
    return _tc_relayout(_sc_kernel(d, t))


# speedup vs baseline: 2.2275x; 2.2275x over previous
import functools

import jax
import jax.numpy as jnp
from jax import lax
from jax.experimental import pallas as pl
from jax.experimental.pallas import tpu as pltpu
from jax.experimental.pallas import tpu_sc as plsc

EMBED = 64
ROWS = 65
N = 16384 * 50
NW = 32
N_TILE = N // NW
CHUNK = 256
NCHUNK = N_TILE // CHUNK


def _sc_body(dist_hbm, table_hbm, out_hbm, dist_v, table_v, ptab_v, out_v0,
             out_v1, sem0, sem1):
    wid = lax.axis_index("s") * 2 + lax.axis_index("c")
    base = wid * N_TILE
    pltpu.sync_copy(table_hbm, table_v)
    pltpu.sync_copy(dist_hbm.at[pl.ds(base, N_TILE)], dist_v)

    def pack_body(k, c2):
        for c in range(4):
            lo = table_v[pl.ds(k * EMBED + c * 16, 16)]
            hi = table_v[pl.ds(k * EMBED + EMBED + c * 16, 16)]
            dl = hi - lo
            lob = plsc.bitcast(lo, jnp.int32)
            dlb = plsc.bitcast(dl, jnp.int32)
            w = ((dlb + 0x8000) & jnp.int32(-65536)) | (
                ((lob + 0x8000) >> 16) & 0xFFFF)
            ptab_v[pl.ds(k * EMBED + c * 16, 16)] = w
        return c2

    lax.fori_loop(0, EMBED, pack_body, 0)

    def compute_chunk(off, out_v):
        def grp_body(j, c2):
            d = dist_v[pl.ds(off + j * 16, 16)]
            f = d * 64.0
            l = f.astype(jnp.int32)
            frac = f - l.astype(jnp.float32)
            li = l * EMBED
            for k0 in range(0, 16, 8):
                bs = [li[k0 + t] for t in range(8)]
                rs = [[ptab_v[pl.ds(b + c * 16, 16)] for c in range(4)]
                      for b in bs]
                fs = [jnp.full((16,), frac[k0 + t], jnp.float32)
                      for t in range(8)]
                for t in range(8):
                    for c in range(4):
                        w = rs[t][c]
                        lo = plsc.bitcast(w << 16, jnp.float32)
                        dl = plsc.bitcast(w & jnp.int32(-65536),
                                          jnp.float32)
                        out_v[j * 16 + k0 + t, pl.ds(c * 16, 16)] = (
                            lo + fs[t] * dl)
            return c2

        lax.fori_loop(0, CHUNK // 16, grp_body, 0)

    def pair_body(gg, carry):
        for buf, sem in ((out_v0, sem0), (out_v1, sem1)):
            g = gg * 2 + (0 if buf is out_v0 else 1)
            off = g * CHUNK
            dst = out_hbm.at[pl.ds(base + off, CHUNK)]

            @pl.when(gg > 0)
            def _wait():
                prev = out_hbm.at[pl.ds(base + off - 2 * CHUNK, CHUNK)]
                pltpu.make_async_copy(buf, prev, sem).wait()

            compute_chunk(off, buf)
            pltpu.async_copy(buf, dst, sem)
        return carry

    lax.fori_loop(0, NCHUNK // 2, pair_body, 0)
    last0 = out_hbm.at[pl.ds(base + (NCHUNK - 2) * CHUNK, CHUNK)]
    last1 = out_hbm.at[pl.ds(base + (NCHUNK - 1) * CHUNK, CHUNK)]
    pltpu.make_async_copy(out_v0, last0, sem0).wait()
    pltpu.make_async_copy(out_v1, last1, sem1).wait()


_sc_kernel = functools.partial(
    pl.kernel,
    out_type=jax.ShapeDtypeStruct((N, EMBED), jnp.float32),
    mesh=plsc.VectorSubcoreMesh(core_axis_name="c", subcore_axis_name="s"),
    compiler_params=pltpu.CompilerParams(needs_layout_passes=False),
    scratch_types=[
        pltpu.VMEM((N_TILE,), jnp.float32),
        pltpu.VMEM((ROWS * EMBED,), jnp.float32),
        pltpu.VMEM((EMBED * EMBED,), jnp.int32),
        pltpu.VMEM((CHUNK, EMBED), jnp.float32),
        pltpu.VMEM((CHUNK, EMBED), jnp.float32),
        pltpu.SemaphoreType.DMA,
        pltpu.SemaphoreType.DMA,
    ],
)(_sc_body)


def kernel(dist, embed_q_weight):
    d = dist.reshape(-1).astype(jnp.float32)
    t = embed_q_weight.reshape(-1)
    return _sc_kernel(d, t)
